# baseline (speedup 1.0000x reference)
import functools

import jax
import jax.numpy as jnp
from jax import lax
from jax.experimental import pallas as pl
from jax.experimental.pallas import tpu as pltpu
from jax.experimental.pallas import tpu_sc as plsc

DIM = 64
PDIM = 128
NC = 2
NS = 16
NW = NC * NS

NBG = 8


def _emb_lookup(idx, table):
    mesh = plsc.VectorSubcoreMesh(core_axis_name="c", subcore_axis_name="s")
    nbatch, seq = idx.shape
    b_per_w = nbatch // NW
    n_groups = b_per_w // NBG

    @functools.partial(
        pl.kernel,
        mesh=mesh,
        out_type=jax.ShapeDtypeStruct((nbatch, seq, PDIM), jnp.float32),
        scratch_types=[
            pltpu.VMEM((2, NBG, seq), jnp.int32),
            pltpu.VMEM((2, NBG, seq, PDIM), jnp.float32),
            pltpu.SemaphoreType.DMA,
            pltpu.SemaphoreType.DMA,
        ],
    )
    def body(idx_hbm, table_hbm, out_hbm, idx_v, rows_v, sem0, sem1):
        wid = lax.axis_index("s") * NC + lax.axis_index("c")
        base_b = wid * b_per_w
        sems = (sem0, sem1)

        def stage_and_fire(g, buf):
            pltpu.sync_copy(
                idx_hbm.at[pl.ds(base_b + g * NBG, NBG)], idx_v.at[buf]
            )
            sem = sems[buf]
            for i in range(NBG):
                pltpu.async_copy(
                    table_hbm.at[idx_v.at[buf, i]],
                    rows_v.at[buf, i],
                    sem,
                )

        def drain_and_store(g, buf):
            sem = sems[buf]
            for i in range(NBG):
                pltpu.make_async_copy(
                    table_hbm.at[idx_v.at[buf, i]], rows_v.at[buf, i], sem
                ).wait()
            pltpu.sync_copy(
                rows_v.at[buf], out_hbm.at[pl.ds(base_b + g * NBG, NBG)]
            )

        stage_and_fire(0, 0)
        n_outer = n_groups // 2

        def outer(t, carry):
            g0 = 2 * t
            stage_and_fire(g0 + 1, 1)
            drain_and_store(g0, 0)

            @pl.when(t + 1 < n_outer)
            def _():
                stage_and_fire(g0 + 2, 0)

            drain_and_store(g0 + 1, 1)
            return carry

        lax.fori_loop(0, n_outer, outer, 0)

    return body(idx, table)


TBLK = 1024


def _tpad_body(emb_t_ref, out_ref):
    out_ref[:, :DIM] = emb_t_ref[...].T


def _transpose_pad(emb_t):
    nrows = emb_t.shape[1]
    grid = (nrows + TBLK - 1) // TBLK
    return pl.pallas_call(
        _tpad_body,
        grid=(grid,),
        in_specs=[pl.BlockSpec((DIM, TBLK), lambda i: (0, i))],
        out_specs=pl.BlockSpec((TBLK, PDIM), lambda i: (i, 0)),
        out_shape=jax.ShapeDtypeStruct((nrows, PDIM), jnp.float32),
    )(emb_t)


def kernel(token_ids, embeddings):
    table_p = _transpose_pad(embeddings.T)
    out_p = _emb_lookup(token_ids.astype(jnp.int32), table_p)
    return out_p[:, :, :DIM]


# speedup vs baseline: 1.5828x; 1.5828x over previous
import functools

import jax
import jax.numpy as jnp
from jax import lax
from jax.experimental import pallas as pl
from jax.experimental.pallas import tpu as pltpu
from jax.experimental.pallas import tpu_sc as plsc

DIM = 64
PDIM = 128
NC = 2
NS = 16
NW = NC * NS

NBG = 8


def _emb_lookup(idx, table):
    mesh = plsc.VectorSubcoreMesh(core_axis_name="c", subcore_axis_name="s")
    nbatch, seq = idx.shape
    b_per_w = nbatch // NW
    n_groups = b_per_w // NBG

    @functools.partial(
        pl.kernel,
        mesh=mesh,
        out_type=jax.ShapeDtypeStruct((nbatch, seq, PDIM), jnp.float32),
        scratch_types=[
            pltpu.VMEM((2, NBG, seq), jnp.int32),
            pltpu.VMEM((2, NBG, seq, PDIM), jnp.float32),
            pltpu.SemaphoreType.DMA,
            pltpu.SemaphoreType.DMA,
        ],
    )
    def body(idx_hbm, table_hbm, out_hbm, idx_v, rows_v, sem0, sem1):
        wid = lax.axis_index("s") * NC + lax.axis_index("c")
        base_b = wid * b_per_w
        sems = (sem0, sem1)

        def stage_and_fire(g, buf):
            pltpu.sync_copy(
                idx_hbm.at[pl.ds(base_b + g * NBG, NBG)], idx_v.at[buf]
            )
            sem = sems[buf]
            for i in range(NBG):
                pltpu.async_copy(
                    table_hbm.at[idx_v.at[buf, i]],
                    rows_v.at[buf, i],
                    sem,
                )

        def drain_and_store(g, buf):
            sem = sems[buf]
            for i in range(NBG):
                pltpu.make_async_copy(
                    table_hbm.at[idx_v.at[buf, i]], rows_v.at[buf, i], sem
                ).wait()
            pltpu.sync_copy(
                rows_v.at[buf], out_hbm.at[pl.ds(base_b + g * NBG, NBG)]
            )

        stage_and_fire(0, 0)
        n_outer = n_groups // 2

        def outer(t, carry):
            g0 = 2 * t
            stage_and_fire(g0 + 1, 1)
            drain_and_store(g0, 0)

            @pl.when(t + 1 < n_outer)
            def _():
                stage_and_fire(g0 + 2, 0)

            drain_and_store(g0 + 1, 1)
            return carry

        lax.fori_loop(0, n_outer, outer, 0)

    return body(idx, table)


TBLK = 8192


def _tpad_body(emb_t_ref, out_ref):
    out_ref[:, :DIM] = emb_t_ref[...].T


def _transpose_pad(emb_t):
    nrows = emb_t.shape[1]
    grid = (nrows + TBLK - 1) // TBLK
    return pl.pallas_call(
        _tpad_body,
        grid=(grid,),
        in_specs=[pl.BlockSpec((DIM, TBLK), lambda i: (0, i))],
        out_specs=pl.BlockSpec((TBLK, PDIM), lambda i: (i, 0)),
        out_shape=jax.ShapeDtypeStruct((nrows, PDIM), jnp.float32),
    )(emb_t)


def kernel(token_ids, embeddings):
    table_p = _transpose_pad(embeddings.T)
    out_p = _emb_lookup(token_ids.astype(jnp.int32), table_p)
    return out_p[:, :, :DIM]


# speedup vs baseline: 1.6130x; 1.0191x over previous
import functools

import jax
import jax.numpy as jnp
from jax import lax
from jax.experimental import pallas as pl
from jax.experimental.pallas import tpu as pltpu
from jax.experimental.pallas import tpu_sc as plsc

DIM = 64
PDIM = 128
NC = 2
NS = 16
NW = NC * NS

NBG = 8


def _emb_lookup(idx, table):
    mesh = plsc.VectorSubcoreMesh(core_axis_name="c", subcore_axis_name="s")
    nbatch, seq = idx.shape
    b_per_w = nbatch // NW
    n_groups = b_per_w // NBG

    @functools.partial(
        pl.kernel,
        mesh=mesh,
        out_type=jax.ShapeDtypeStruct((nbatch, seq, PDIM), jnp.float32),
        scratch_types=[
            pltpu.VMEM((2, NBG, seq), jnp.int32),
            pltpu.VMEM((2, NBG, seq, PDIM), jnp.float32),
            pltpu.SemaphoreType.DMA,
            pltpu.SemaphoreType.DMA,
        ],
    )
    def body(idx_hbm, table_hbm, out_hbm, idx_v, rows_v, sem0, sem1):
        wid = lax.axis_index("s") * NC + lax.axis_index("c")
        base_b = wid * b_per_w
        sems = (sem0, sem1)

        def stage_and_fire(g, buf):
            pltpu.sync_copy(
                idx_hbm.at[pl.ds(base_b + g * NBG, NBG)], idx_v.at[buf]
            )
            sem = sems[buf]
            for i in range(NBG):
                pltpu.async_copy(
                    table_hbm.at[idx_v.at[buf, i]],
                    rows_v.at[buf, i],
                    sem,
                )

        def drain_and_store(g, buf):
            sem = sems[buf]
            for i in range(NBG):
                pltpu.make_async_copy(
                    table_hbm.at[idx_v.at[buf, i]], rows_v.at[buf, i], sem
                ).wait()
            pltpu.sync_copy(
                rows_v.at[buf], out_hbm.at[pl.ds(base_b + g * NBG, NBG)]
            )

        stage_and_fire(0, 0)
        n_outer = n_groups // 2

        def outer(t, carry):
            g0 = 2 * t
            stage_and_fire(g0 + 1, 1)
            drain_and_store(g0, 0)

            @pl.when(t + 1 < n_outer)
            def _():
                stage_and_fire(g0 + 2, 0)

            drain_and_store(g0 + 1, 1)
            return carry

        lax.fori_loop(0, n_outer, outer, 0)

    return body(idx, table)


TBLK = 16384


def _tpad_body(emb_t_ref, out_ref):
    out_ref[:, :DIM] = emb_t_ref[...].T


def _transpose_pad(emb_t):
    nrows = emb_t.shape[1]
    grid = (nrows + TBLK - 1) // TBLK
    return pl.pallas_call(
        _tpad_body,
        grid=(grid,),
        in_specs=[pl.BlockSpec((DIM, TBLK), lambda i: (0, i))],
        out_specs=pl.BlockSpec((TBLK, PDIM), lambda i: (i, 0)),
        out_shape=jax.ShapeDtypeStruct((nrows, PDIM), jnp.float32),
    )(emb_t)


def kernel(token_ids, embeddings):
    table_p = _transpose_pad(embeddings.T)
    out_p = _emb_lookup(token_ids.astype(jnp.int32), table_p)
    return out_p[:, :, :DIM]


# speedup vs baseline: 1.6271x; 1.0087x over previous
import functools

import jax
import jax.numpy as jnp
from jax import lax
from jax.experimental import pallas as pl
from jax.experimental.pallas import tpu as pltpu
from jax.experimental.pallas import tpu_sc as plsc

DIM = 64
PDIM = 128
NC = 2
NS = 16
NW = NC * NS

NBG = 8


def _emb_lookup(idx, table):
    mesh = plsc.VectorSubcoreMesh(core_axis_name="c", subcore_axis_name="s")
    nbatch, seq = idx.shape
    b_per_w = nbatch // NW
    n_groups = b_per_w // NBG

    @functools.partial(
        pl.kernel,
        mesh=mesh,
        out_type=jax.ShapeDtypeStruct((nbatch, seq, PDIM), jnp.float32),
        scratch_types=[
            pltpu.VMEM((2, NBG, seq), jnp.int32),
            pltpu.VMEM((2, NBG, seq, PDIM), jnp.float32),
            pltpu.SemaphoreType.DMA,
            pltpu.SemaphoreType.DMA,
        ],
    )
    def body(idx_hbm, table_hbm, out_hbm, idx_v, rows_v, sem0, sem1):
        wid = lax.axis_index("s") * NC + lax.axis_index("c")
        base_b = wid * b_per_w
        sems = (sem0, sem1)

        def stage_and_fire(g, buf):
            pltpu.sync_copy(
                idx_hbm.at[pl.ds(base_b + g * NBG, NBG)], idx_v.at[buf]
            )
            sem = sems[buf]
            for i in range(NBG):
                pltpu.async_copy(
                    table_hbm.at[idx_v.at[buf, i]],
                    rows_v.at[buf, i],
                    sem,
                )

        def drain_and_store(g, buf):
            sem = sems[buf]
            for i in range(NBG):
                pltpu.make_async_copy(
                    table_hbm.at[idx_v.at[buf, i]], rows_v.at[buf, i], sem
                ).wait()
            pltpu.sync_copy(
                rows_v.at[buf], out_hbm.at[pl.ds(base_b + g * NBG, NBG)]
            )

        stage_and_fire(0, 0)
        n_outer = n_groups // 2

        def outer(t, carry):
            g0 = 2 * t
            stage_and_fire(g0 + 1, 1)
            drain_and_store(g0, 0)

            @pl.when(t + 1 < n_outer)
            def _():
                stage_and_fire(g0 + 2, 0)

            drain_and_store(g0 + 1, 1)
            return carry

        lax.fori_loop(0, n_outer, outer, 0)

    return body(idx, table)


TBLK = 32768


def _tpad_body(emb_t_ref, out_ref):
    out_ref[:, :DIM] = emb_t_ref[...].T


def _transpose_pad(emb_t):
    nrows = emb_t.shape[1]
    grid = (nrows + TBLK - 1) // TBLK
    return pl.pallas_call(
        _tpad_body,
        grid=(grid,),
        in_specs=[pl.BlockSpec((DIM, TBLK), lambda i: (0, i))],
        out_specs=pl.BlockSpec((TBLK, PDIM), lambda i: (i, 0)),
        out_shape=jax.ShapeDtypeStruct((nrows, PDIM), jnp.float32),
    )(emb_t)


def kernel(token_ids, embeddings):
    table_p = _transpose_pad(embeddings.T)
    out_p = _emb_lookup(token_ids.astype(jnp.int32), table_p)
    return out_p[:, :, :DIM]
